# baseline (speedup 1.0000x reference)
import functools

import jax
import jax.numpy as jnp
from jax import lax
from jax.experimental import pallas as pl
from jax.experimental.pallas import tpu as pltpu
from jax.experimental.pallas import tpu_sc as plsc

N = 10000
E = 320000
D = 128
L = 3
K = 3
DH = D // 2

NSUB = 16
NCORE = 2
B = 64
C = 16
NBUF = 8
NCHUNK = 20
E_PAD = NSUB * NCHUNK * C * B
ACC_ROWS = N + 16
ZROWS = ACC_ROWS // NSUB
HROWS = N // NSUB

_BCAST_DNUMS = lax.GatherDimensionNumbers(
    offset_dims=(), collapsed_slice_dims=(0,), start_index_map=(0,))


def _bcast_lane(v16, l):
    idx = jnp.full((16, 1), l, jnp.int32)
    return lax.gather(v16, idx, _BCAST_DNUMS, (1,),
                      mode=lax.GatherScatterMode.PROMISE_IN_BOUNDS)


def _mm_body(x_ref, wlo_ref, whi_ref, blo_ref, bhi_ref, o_ref):
    xb = x_ref[...]
    o_ref[0] = jnp.dot(xb, wlo_ref[...], preferred_element_type=jnp.float32) + blo_ref[...]
    o_ref[1] = jnp.dot(xb, whi_ref[...], preferred_element_type=jnp.float32) + bhi_ref[...]


def _mm(x, wlo, whi, blo, bhi):
    bn = 400
    return pl.pallas_call(
        _mm_body,
        grid=(N // bn,),
        in_specs=[
            pl.BlockSpec((bn, D), lambda i: (i, 0)),
            pl.BlockSpec((D, DH), lambda i: (0, 0)),
            pl.BlockSpec((D, DH), lambda i: (0, 0)),
            pl.BlockSpec((1, DH), lambda i: (0, 0)),
            pl.BlockSpec((1, DH), lambda i: (0, 0)),
        ],
        out_specs=pl.BlockSpec((2, bn, DH), lambda i: (0, i, 0)),
        out_shape=jax.ShapeDtypeStruct((2, N, DH), jnp.float32),
    )(x, wlo, whi, blo, bhi)


def _upd_body(x_ref, alo, ahi, o_ref):
    acc = jnp.concatenate([alo[0], ahi[0]], axis=-1)
    xn = x_ref[...] + jnp.maximum(acc, 0.0)
    nrm = jnp.sqrt(jnp.sum(xn * xn, axis=1, keepdims=True))
    o_ref[...] = xn / jnp.maximum(nrm, 1e-12)


def _upd(x, acc):
    bn = 400
    return pl.pallas_call(
        _upd_body,
        grid=(N // bn,),
        in_specs=[
            pl.BlockSpec((bn, D), lambda i: (i, 0)),
            pl.BlockSpec((1, bn, DH), lambda i: (0, i, 0)),
            pl.BlockSpec((1, bn, DH), lambda i: (1, i, 0)),
        ],
        out_specs=pl.BlockSpec((bn, D), lambda i: (i, 0)),
        out_shape=jax.ShapeDtypeStruct((N, D), jnp.float32),
    )(x, acc, acc)


def _updmm_body(x_ref, alo, ahi, wlo_ref, whi_ref, blo_ref, bhi_ref,
                ox_ref, oh_ref):
    acc = jnp.concatenate([alo[0], ahi[0]], axis=-1)
    xn = x_ref[...] + jnp.maximum(acc, 0.0)
    nrm = jnp.sqrt(jnp.sum(xn * xn, axis=1, keepdims=True))
    xn = xn / jnp.maximum(nrm, 1e-12)
    ox_ref[...] = xn
    oh_ref[0] = jnp.dot(xn, wlo_ref[...], preferred_element_type=jnp.float32) + blo_ref[...]
    oh_ref[1] = jnp.dot(xn, whi_ref[...], preferred_element_type=jnp.float32) + bhi_ref[...]


def _updmm(x, acc, wlo, whi, blo, bhi):
    bn = 400
    return pl.pallas_call(
        _updmm_body,
        grid=(N // bn,),
        in_specs=[
            pl.BlockSpec((bn, D), lambda i: (i, 0)),
            pl.BlockSpec((1, bn, DH), lambda i: (0, i, 0)),
            pl.BlockSpec((1, bn, DH), lambda i: (1, i, 0)),
            pl.BlockSpec((D, DH), lambda i: (0, 0)),
            pl.BlockSpec((D, DH), lambda i: (0, 0)),
            pl.BlockSpec((1, DH), lambda i: (0, 0)),
            pl.BlockSpec((1, DH), lambda i: (0, 0)),
        ],
        out_specs=[
            pl.BlockSpec((bn, D), lambda i: (i, 0)),
            pl.BlockSpec((2, bn, DH), lambda i: (0, i, 0)),
        ],
        out_shape=[
            jax.ShapeDtypeStruct((N, D), jnp.float32),
            jax.ShapeDtypeStruct((2, N, DH), jnp.float32),
        ],
    )(x, acc, acc, wlo, whi, blo, bhi)


def _sc_scatter(h2f, srcp, dstp, attrp, atab, zeros):
    mesh = plsc.VectorSubcoreMesh(
        core_axis_name="c", subcore_axis_name="s", num_cores=NCORE)

    @functools.partial(
        pl.kernel,
        mesh=mesh,
        compiler_params=pltpu.CompilerParams(use_tc_tiling_on_sc=False),
        out_type=jax.ShapeDtypeStruct((NCORE, ACC_ROWS, DH), jnp.float32),
        scratch_types=[
            pltpu.VMEM((2, C, B), jnp.int32),
            pltpu.VMEM((2, C, B), jnp.int32),
            pltpu.VMEM((C, B), jnp.int32),
            pltpu.VMEM((C, B // 16, 16), jnp.float32),
            pltpu.VMEM((16,), jnp.float32),
        ]
        + [pltpu.VMEM((B, DH), jnp.float32) for _ in range(NBUF)]
        + [
            pltpu.VMEM_SHARED((N, DH), jnp.float32),
            pltpu.VMEM_SHARED((ACC_ROWS, DH), jnp.float32),
        ]
        + [pltpu.SemaphoreType.DMA for _ in range(2 * NBUF + 1)],
    )
    def k(h2_hbm, src_hbm, dst_hbm, attr_hbm, atab_hbm, z_hbm, acc_hbm,
          gidx_v, sidx_v, attr_v, wbuf_v, atab_v, *rest):
        rbufs = rest[:NBUF]
        h_sh = rest[NBUF]
        acc_sh = rest[NBUF + 1]
        gsems = rest[NBUF + 2:2 * NBUF + 2]
        ssems = rest[2 * NBUF + 2:3 * NBUF + 2]
        isem = rest[3 * NBUF + 2]
        c = lax.axis_index("c")
        s = lax.axis_index("s")
        pltpu.sync_copy(h2_hbm.at[pl.ds(c * N + s * HROWS, HROWS)],
                        h_sh.at[pl.ds(s * HROWS, HROWS)])
        pltpu.sync_copy(z_hbm, acc_sh.at[pl.ds(s * ZROWS, ZROWS)])
        pltpu.sync_copy(atab_hbm, atab_v)
        pltpu.sync_copy(src_hbm.at[s, 0], gidx_v.at[0])
        pltpu.sync_copy(dst_hbm.at[s, 0], sidx_v.at[0])
        pltpu.sync_copy(attr_hbm.at[s, 0], attr_v)
        plsc.subcore_barrier()
        atab16 = atab_v[pl.ds(0, 16)]
        a1v = _bcast_lane(atab16, 1)
        a2v = _bcast_lane(atab16, 2)
        a3v = _bcast_lane(atab16, 3)

        def chunk(ch, carry):
            a = lax.rem(ch, 2)
            @pl.when(ch > 0)
            def _():
                for p in range(NBUF):
                    pltpu.make_async_copy(
                        rbufs[p], acc_sh.at[sidx_v.at[a, C - NBUF + p]],
                        ssems[p]).wait()
                pltpu.make_async_copy(
                    src_hbm.at[s, ch], gidx_v.at[a], isem).wait()
                pltpu.make_async_copy(
                    dst_hbm.at[s, ch], sidx_v.at[a], isem).wait()
                pltpu.make_async_copy(
                    attr_hbm.at[s, ch], attr_v, isem).wait()

            for j in range(C):
                for q in range(B // 16):
                    sl = pl.ds(q * 16, 16)
                    at16 = attr_v[j, sl]
                    wbuf_v[j, q] = jnp.where(
                        at16 == 1, a1v, jnp.where(at16 == 2, a2v, a3v))

            @pl.when(ch < NCHUNK - 1)
            def _():
                pltpu.async_copy(src_hbm.at[s, ch + 1], gidx_v.at[1 - a], isem)
                pltpu.async_copy(dst_hbm.at[s, ch + 1], sidx_v.at[1 - a], isem)
                pltpu.async_copy(attr_hbm.at[s, ch + 1], attr_v, isem)

            for p in range(NBUF):
                pltpu.async_copy(h_sh.at[gidx_v.at[a, p]], rbufs[p], gsems[p])
            for j in range(C):
                p = j % NBUF
                pltpu.make_async_copy(
                    h_sh.at[gidx_v.at[a, j]], rbufs[p], gsems[p]).wait()

                @plsc.parallel_loop(0, B // 16, unroll=2)
                def scale(bi):
                    w16 = wbuf_v[j, bi]
                    for l in range(16):
                        wb = _bcast_lane(w16, l)
                        e = bi * 16 + l
                        for q in range(DH // 16):
                            sl = pl.ds(q * 16, 16)
                            rbufs[p][e, sl] = rbufs[p][e, sl] * wb
                pltpu.async_copy(rbufs[p], acc_sh.at[sidx_v.at[a, j]],
                                 ssems[p], add=True)
                if j + NBUF < C:
                    pltpu.make_async_copy(
                        rbufs[p], acc_sh.at[sidx_v.at[a, j]], ssems[p]).wait()
                    pltpu.async_copy(h_sh.at[gidx_v.at[a, j + NBUF]],
                                     rbufs[p], gsems[p])
            return carry

        lax.fori_loop(0, NCHUNK, chunk, 0)
        last = (NCHUNK - 1) % 2
        for p in range(NBUF):
            pltpu.make_async_copy(
                rbufs[p], acc_sh.at[sidx_v.at[last, C - NBUF + p]],
                ssems[p]).wait()
        plsc.subcore_barrier()
        pltpu.sync_copy(acc_sh.at[pl.ds(s * ZROWS, ZROWS)],
                        acc_hbm.at[c, pl.ds(s * ZROWS, ZROWS)])

    return k(h2f, srcp, dstp, attrp, atab, zeros)


def kernel(x, edge_index, edge_attr, alpha, W, b):
    x = x.astype(jnp.float32)
    src = edge_index[0].astype(jnp.int32)
    dst = edge_index[1].astype(jnp.int32)
    attr = edge_attr.astype(jnp.int32)
    pad = E_PAD - E
    srcp = jnp.concatenate([src, jnp.zeros((pad,), jnp.int32)]).reshape(
        NSUB, NCHUNK, C, B)
    dstp = jnp.concatenate([dst, jnp.full((pad,), N, jnp.int32)]).reshape(
        NSUB, NCHUNK, C, B)
    attrp = jnp.concatenate([attr, jnp.full((pad,), K, jnp.int32)]).reshape(
        NSUB, NCHUNK, C, B)
    zeros = jnp.zeros((ZROWS, DH), jnp.float32)
    a = jax.nn.softmax(alpha.astype(jnp.float32))
    atab = jnp.zeros((16,), jnp.float32).at[1:1 + K].set(a)

    wlo = [W[t, :, :DH].astype(jnp.float32) for t in range(L)]
    whi = [W[t, :, DH:].astype(jnp.float32) for t in range(L)]
    blo = [b[t, :DH].astype(jnp.float32).reshape(1, DH) for t in range(L)]
    bhi = [b[t, DH:].astype(jnp.float32).reshape(1, DH) for t in range(L)]

    h2 = _mm(x, wlo[0], whi[0], blo[0], bhi[0])
    for t in range(L):
        acc = _sc_scatter(h2.reshape(2 * N, DH), srcp, dstp, attrp,
                          atab, zeros)
        if t + 1 < L:
            x, h2 = _updmm(x, acc, wlo[t + 1], whi[t + 1],
                           blo[t + 1], bhi[t + 1])
        else:
            x = _upd(x, acc)
    return x


# speedup vs baseline: 1.1591x; 1.1591x over previous
import functools

import jax
import jax.numpy as jnp
from jax import lax
from jax.experimental import pallas as pl
from jax.experimental.pallas import tpu as pltpu
from jax.experimental.pallas import tpu_sc as plsc

N = 10000
E = 320000
D = 128
L = 3
K = 3
DH = D // 2

NSUB = 16
NCORE = 2
B = 64
C = 8
NCHUNK = 40
E_PAD = NSUB * NCHUNK * C * B
ACC_ROWS = N + 16
ZROWS = ACC_ROWS // NSUB
HROWS = N // NSUB

_BCAST_DNUMS = lax.GatherDimensionNumbers(
    offset_dims=(), collapsed_slice_dims=(0,), start_index_map=(0,))


def _bcast_lane(v16, l):
    idx = jnp.full((16, 1), l, jnp.int32)
    return lax.gather(v16, idx, _BCAST_DNUMS, (1,),
                      mode=lax.GatherScatterMode.PROMISE_IN_BOUNDS)


def _mm_body(x_ref, wlo_ref, whi_ref, blo_ref, bhi_ref, o_ref):
    xb = x_ref[...]
    o_ref[0] = jnp.dot(xb, wlo_ref[...], preferred_element_type=jnp.float32) + blo_ref[...]
    o_ref[1] = jnp.dot(xb, whi_ref[...], preferred_element_type=jnp.float32) + bhi_ref[...]


def _mm(x, wlo, whi, blo, bhi):
    bn = 1000
    return pl.pallas_call(
        _mm_body,
        grid=(N // bn,),
        in_specs=[
            pl.BlockSpec((bn, D), lambda i: (i, 0)),
            pl.BlockSpec((D, DH), lambda i: (0, 0)),
            pl.BlockSpec((D, DH), lambda i: (0, 0)),
            pl.BlockSpec((1, DH), lambda i: (0, 0)),
            pl.BlockSpec((1, DH), lambda i: (0, 0)),
        ],
        out_specs=pl.BlockSpec((2, bn, DH), lambda i: (0, i, 0)),
        out_shape=jax.ShapeDtypeStruct((2, N, DH), jnp.float32),
    )(x, wlo, whi, blo, bhi)


def _upd_body(x_ref, alo, ahi, o_ref):
    acc = jnp.concatenate([alo[0], ahi[0]], axis=-1)
    xn = x_ref[...] + jnp.maximum(acc, 0.0)
    nrm = jnp.sqrt(jnp.sum(xn * xn, axis=1, keepdims=True))
    o_ref[...] = xn / jnp.maximum(nrm, 1e-12)


def _upd(x, acc):
    bn = 1000
    return pl.pallas_call(
        _upd_body,
        grid=(N // bn,),
        in_specs=[
            pl.BlockSpec((bn, D), lambda i: (i, 0)),
            pl.BlockSpec((1, bn, DH), lambda i: (0, i, 0)),
            pl.BlockSpec((1, bn, DH), lambda i: (1, i, 0)),
        ],
        out_specs=pl.BlockSpec((bn, D), lambda i: (i, 0)),
        out_shape=jax.ShapeDtypeStruct((N, D), jnp.float32),
    )(x, acc, acc)


def _updmm_body(x_ref, alo, ahi, wlo_ref, whi_ref, blo_ref, bhi_ref,
                ox_ref, oh_ref):
    acc = jnp.concatenate([alo[0], ahi[0]], axis=-1)
    xn = x_ref[...] + jnp.maximum(acc, 0.0)
    nrm = jnp.sqrt(jnp.sum(xn * xn, axis=1, keepdims=True))
    xn = xn / jnp.maximum(nrm, 1e-12)
    ox_ref[...] = xn
    oh_ref[0] = jnp.dot(xn, wlo_ref[...], preferred_element_type=jnp.float32) + blo_ref[...]
    oh_ref[1] = jnp.dot(xn, whi_ref[...], preferred_element_type=jnp.float32) + bhi_ref[...]


def _updmm(x, acc, wlo, whi, blo, bhi):
    bn = 1000
    return pl.pallas_call(
        _updmm_body,
        grid=(N // bn,),
        in_specs=[
            pl.BlockSpec((bn, D), lambda i: (i, 0)),
            pl.BlockSpec((1, bn, DH), lambda i: (0, i, 0)),
            pl.BlockSpec((1, bn, DH), lambda i: (1, i, 0)),
            pl.BlockSpec((D, DH), lambda i: (0, 0)),
            pl.BlockSpec((D, DH), lambda i: (0, 0)),
            pl.BlockSpec((1, DH), lambda i: (0, 0)),
            pl.BlockSpec((1, DH), lambda i: (0, 0)),
        ],
        out_specs=[
            pl.BlockSpec((bn, D), lambda i: (i, 0)),
            pl.BlockSpec((2, bn, DH), lambda i: (0, i, 0)),
        ],
        out_shape=[
            jax.ShapeDtypeStruct((N, D), jnp.float32),
            jax.ShapeDtypeStruct((2, N, DH), jnp.float32),
        ],
    )(x, acc, acc, wlo, whi, blo, bhi)


def _sc_scatter(h2f, srcp, dstp, attrp, atab, zeros):
    mesh = plsc.VectorSubcoreMesh(
        core_axis_name="c", subcore_axis_name="s", num_cores=NCORE)

    @functools.partial(
        pl.kernel,
        mesh=mesh,
        compiler_params=pltpu.CompilerParams(use_tc_tiling_on_sc=False),
        out_type=jax.ShapeDtypeStruct((NCORE, ACC_ROWS, DH), jnp.float32),
        scratch_types=[
            pltpu.VMEM((2, C, B), jnp.int32),
            pltpu.VMEM((2, C, B), jnp.int32),
            pltpu.VMEM((C, B), jnp.int32),
            pltpu.VMEM((C, B // 16, 16), jnp.float32),
            pltpu.VMEM((16,), jnp.float32),
        ]
        + [pltpu.VMEM((B, DH), jnp.float32) for _ in range(C)]
        + [
            pltpu.VMEM_SHARED((N, DH), jnp.float32),
            pltpu.VMEM_SHARED((ACC_ROWS, DH), jnp.float32),
        ]
        + [pltpu.SemaphoreType.DMA for _ in range(2 * C + 1)],
    )
    def k(h2_hbm, src_hbm, dst_hbm, attr_hbm, atab_hbm, z_hbm, acc_hbm,
          gidx_v, sidx_v, attr_v, wbuf_v, atab_v, *rest):
        rbufs = rest[:C]
        h_sh = rest[C]
        acc_sh = rest[C + 1]
        gsems = rest[C + 2:2 * C + 2]
        ssems = rest[2 * C + 2:3 * C + 2]
        isem = rest[3 * C + 2]
        c = lax.axis_index("c")
        s = lax.axis_index("s")
        pltpu.sync_copy(h2_hbm.at[pl.ds(c * N + s * HROWS, HROWS)],
                        h_sh.at[pl.ds(s * HROWS, HROWS)])
        pltpu.sync_copy(z_hbm, acc_sh.at[pl.ds(s * ZROWS, ZROWS)])
        pltpu.sync_copy(atab_hbm, atab_v)
        pltpu.sync_copy(src_hbm.at[s, 0], gidx_v.at[0])
        pltpu.sync_copy(dst_hbm.at[s, 0], sidx_v.at[0])
        pltpu.sync_copy(attr_hbm.at[s, 0], attr_v)
        plsc.subcore_barrier()
        atab16 = atab_v[pl.ds(0, 16)]
        a1v = _bcast_lane(atab16, 1)
        a2v = _bcast_lane(atab16, 2)
        a3v = _bcast_lane(atab16, 3)

        def chunk(ch, carry):
            a = lax.rem(ch, 2)
            @pl.when(ch > 0)
            def _():
                for p in range(C):
                    pltpu.make_async_copy(
                        rbufs[p], acc_sh.at[sidx_v.at[a, p]], ssems[p]).wait()
                pltpu.make_async_copy(
                    src_hbm.at[s, ch], gidx_v.at[a], isem).wait()
                pltpu.make_async_copy(
                    dst_hbm.at[s, ch], sidx_v.at[a], isem).wait()
                pltpu.make_async_copy(
                    attr_hbm.at[s, ch], attr_v, isem).wait()

            for j in range(C):
                for q in range(B // 16):
                    sl = pl.ds(q * 16, 16)
                    at16 = attr_v[j, sl]
                    wbuf_v[j, q] = jnp.where(
                        at16 == 1, a1v, jnp.where(at16 == 2, a2v, a3v))

            @pl.when(ch < NCHUNK - 1)
            def _():
                pltpu.async_copy(src_hbm.at[s, ch + 1], gidx_v.at[1 - a], isem)
                pltpu.async_copy(dst_hbm.at[s, ch + 1], sidx_v.at[1 - a], isem)
                pltpu.async_copy(attr_hbm.at[s, ch + 1], attr_v, isem)

            for p in range(C):
                pltpu.async_copy(h_sh.at[gidx_v.at[a, p]], rbufs[p], gsems[p])
            for j in range(C):
                pltpu.make_async_copy(
                    h_sh.at[gidx_v.at[a, j]], rbufs[j], gsems[j]).wait()

                @plsc.parallel_loop(0, B // 16, unroll=2)
                def scale(bi):
                    w16 = wbuf_v[j, bi]
                    for l in range(16):
                        wb = _bcast_lane(w16, l)
                        e = bi * 16 + l
                        for q in range(DH // 16):
                            sl = pl.ds(q * 16, 16)
                            rbufs[j][e, sl] = rbufs[j][e, sl] * wb
                pltpu.async_copy(rbufs[j], acc_sh.at[sidx_v.at[a, j]],
                                 ssems[j], add=True)
            return carry

        lax.fori_loop(0, NCHUNK, chunk, 0)
        last = (NCHUNK - 1) % 2
        for p in range(C):
            pltpu.make_async_copy(
                rbufs[p], acc_sh.at[sidx_v.at[last, p]], ssems[p]).wait()
        plsc.subcore_barrier()
        pltpu.sync_copy(acc_sh.at[pl.ds(s * ZROWS, ZROWS)],
                        acc_hbm.at[c, pl.ds(s * ZROWS, ZROWS)])

    return k(h2f, srcp, dstp, attrp, atab, zeros)


def kernel(x, edge_index, edge_attr, alpha, W, b):
    x = x.astype(jnp.float32)
    src = edge_index[0].astype(jnp.int32)
    dst = edge_index[1].astype(jnp.int32)
    attr = edge_attr.astype(jnp.int32)
    pad = E_PAD - E
    srcp = jnp.concatenate([src, jnp.zeros((pad,), jnp.int32)]).reshape(
        NSUB, NCHUNK, C, B)
    dstp = jnp.concatenate([dst, jnp.full((pad,), N, jnp.int32)]).reshape(
        NSUB, NCHUNK, C, B)
    attrp = jnp.concatenate([attr, jnp.full((pad,), K, jnp.int32)]).reshape(
        NSUB, NCHUNK, C, B)
    zeros = jnp.zeros((ZROWS, DH), jnp.float32)
    a = jax.nn.softmax(alpha.astype(jnp.float32))
    atab = jnp.zeros((16,), jnp.float32).at[1:1 + K].set(a)

    wlo = [W[t, :, :DH].astype(jnp.float32) for t in range(L)]
    whi = [W[t, :, DH:].astype(jnp.float32) for t in range(L)]
    blo = [b[t, :DH].astype(jnp.float32).reshape(1, DH) for t in range(L)]
    bhi = [b[t, DH:].astype(jnp.float32).reshape(1, DH) for t in range(L)]

    h2 = _mm(x, wlo[0], whi[0], blo[0], bhi[0])
    for t in range(L):
        acc = _sc_scatter(h2.reshape(2 * N, DH), srcp, dstp, attrp,
                          atab, zeros)
        if t + 1 < L:
            x, h2 = _updmm(x, acc, wlo[t + 1], whi[t + 1],
                           blo[t + 1], bhi[t + 1])
        else:
            x = _upd(x, acc)
    return x
